# baseline (speedup 1.0000x reference)
import functools

import jax
import jax.numpy as jnp
from jax import lax
from jax.experimental import pallas as pl
from jax.experimental.pallas import tpu as pltpu
from jax.experimental.pallas import tpu_sc as plsc

_TAU_WORD = 0.1
_ALPHA = 0.7

_NC = 2
_NS = 16
_L = 16
_NW = _NC * _NS

_GDN = lax.GatherDimensionNumbers(
    offset_dims=(), collapsed_slice_dims=(0,), start_index_map=(0,))


def _permute(x, idx):
    return lax.gather(x, idx[:, None], _GDN, (1,),
                      mode=lax.GatherScatterMode.PROMISE_IN_BOUNDS)


def _lane_total(x):
    iota = lax.iota(jnp.int32, _L)
    for shift in (8, 4, 2, 1):
        x = x + _permute(x, iota ^ shift)
    return x


def _make_tc_kernel(N, V, row0):
    R = 8
    NR = N - row0
    G = NR // R
    SL = V // 128

    def body(t_sref, m_sref, logp_hbm, sim_hbm, out_ref,
             sim_buf, logp_buf, acc_ref, sem_s, sem_l):
        i = pl.program_id(0)
        lg3 = logp_hbm
        sm3 = sim_hbm

        def issue(g, slot):
            row = row0 + g * R
            pltpu.make_async_copy(
                lg3.at[pl.ds(row * SL, R * SL)], logp_buf.at[slot],
                sem_l.at[slot]).start()
            for k in range(R):
                t = t_sref[row + k]
                pltpu.make_async_copy(
                    sm3.at[pl.ds(t * SL, SL)],
                    sim_buf.at[slot, pl.ds(k * SL, SL)],
                    sem_s.at[slot]).start()

        @pl.when(i == 0)
        def _():
            acc_ref[...] = jnp.zeros_like(acc_ref)
            issue(0, 0)

        slot = i % 2

        @pl.when(i + 1 < G)
        def _():
            issue(i + 1, (i + 1) % 2)

        row = row0 + i * R
        pltpu.make_async_copy(
            lg3.at[pl.ds(row * SL, R * SL)], logp_buf.at[slot],
            sem_l.at[slot]).wait()
        pltpu.make_async_copy(
            sm3.at[pl.ds(0, R * SL)], sim_buf.at[slot],
            sem_s.at[slot]).wait()

        lg = logp_buf[slot]
        e = jnp.exp(sim_buf[slot] * jnp.float32(1.0 / _TAU_WORD))
        sel = (lax.broadcasted_iota(jnp.int32, (R, R * SL), 1) // SL
               == lax.broadcasted_iota(jnp.int32, (R, R * SL), 0)
               ).astype(jnp.float32)
        zrow = jax.lax.dot(sel, e, preferred_element_type=jnp.float32)
        drow = jax.lax.dot(sel, e * lg,
                           preferred_element_type=jnp.float32)
        z8 = jnp.sum(zrow, axis=1, keepdims=True)
        d8 = jnp.sum(drow, axis=1, keepdims=True)
        li128 = lax.broadcasted_iota(jnp.int32, (1, 128), 1)
        picks = []
        for k in range(R):
            t = t_sref[row + k]
            stripe = logp_buf[slot, pl.ds(k * SL + t // 128, 1), :]
            picks.append(jnp.sum(jnp.where(li128 == t % 128, stripe, 0.0)))
        pick8 = jnp.stack(picks)[:, None]
        t_m = jnp.stack([m_sref[row + k] for k in range(R)])[:, None]
        li = lax.broadcasted_iota(jnp.int32, (R, 128), 1)
        contrib = jnp.where(
            li == 0, t_m * d8 / z8,
            jnp.where(li == 1, t_m * pick8, jnp.where(li == 2, t_m, 0.0)))
        acc_ref[...] += contrib

        @pl.when(i == G - 1)
        def _():
            out_ref[...] = jnp.sum(acc_ref[...], axis=0, keepdims=True)

    return pl.pallas_call(
        body,
        grid_spec=pltpu.PrefetchScalarGridSpec(
            num_scalar_prefetch=2,
            grid=(G,),
            in_specs=[
                pl.BlockSpec(memory_space=pltpu.HBM),
                pl.BlockSpec(memory_space=pltpu.HBM),
            ],
            out_specs=pl.BlockSpec((1, 128), lambda i, t, m: (0, 0)),
            scratch_shapes=[
                pltpu.VMEM((2, R * SL, 128), jnp.float32),
                pltpu.VMEM((2, R * SL, 128), jnp.float32),
                pltpu.VMEM((R, 128), jnp.float32),
                pltpu.SemaphoreType.DMA((2,)),
                pltpu.SemaphoreType.DMA((2,)),
            ],
        ),
        out_shape=jax.ShapeDtypeStruct((1, 128), jnp.float32),
    )


def _make_sc_kernel(N, V):
    RW = N // _NW
    C = 4
    NCHUNK = RW // C
    VCH = V // _L
    mesh = plsc.VectorSubcoreMesh(core_axis_name="c", subcore_axis_name="s",
                                  num_cores=_NC, num_subcores=_NS)

    @functools.partial(
        pl.kernel,
        mesh=mesh,
        compiler_params=pltpu.CompilerParams(needs_layout_passes=False),
        out_type=jax.ShapeDtypeStruct((_NW, 3, _L), jnp.float32),
        scratch_types=[
            pltpu.VMEM((RW,), jnp.int32),
            pltpu.VMEM((NCHUNK, C), jnp.int32),
            pltpu.VMEM((RW,), jnp.float32),
            pltpu.VMEM((2, C, V), jnp.float32),
            pltpu.VMEM((2, C, V), jnp.float32),
            pltpu.VMEM((3, _L), jnp.float32),
            pltpu.SemaphoreType.DMA,
            pltpu.SemaphoreType.DMA,
        ],
    )
    def sc_kernel(logp_hbm, sim_hbm, tgt_hbm, tgt3_hbm, mask_hbm,
                  out_hbm, idx_v, idx2_v, mask_v, sim_buf,
                  logp_buf, out_stage, sem_sim, sem_logp):
        wid = lax.axis_index("s") * _NC + lax.axis_index("c")
        base = wid * RW

        pltpu.sync_copy(tgt_hbm.at[pl.ds(base, RW)], idx_v)
        pltpu.sync_copy(tgt3_hbm.at[wid], idx2_v)
        pltpu.sync_copy(mask_hbm.at[pl.ds(base, RW)], mask_v)

        iota16 = lax.iota(jnp.int32, _L)
        rowsel = iota16 % C
        firstcopy = (iota16 < C).astype(jnp.float32)

        def start_chunk(c, slot):
            sim = pltpu.async_copy(
                sim_hbm.at[idx2_v.at[c]], sim_buf.at[slot], sem_sim)
            lp = pltpu.async_copy(
                logp_hbm.at[pl.ds(base + c * C, C)], logp_buf.at[slot],
                sem_logp)
            return sim, lp

        smooth_acc = jnp.zeros((_L,), jnp.float32)
        ml_acc = jnp.zeros((_L,), jnp.float32)
        inflight = start_chunk(0, 0)
        inv_tau = jnp.float32(1.0 / _TAU_WORD)

        for c in range(NCHUNK):
            slot = c % 2
            inflight[0].wait()
            inflight[1].wait()
            if c + 1 < NCHUNK:
                inflight = start_chunk(c + 1, (c + 1) % 2)
            chunk_rows = c * C + rowsel
            t_vec = plsc.load_gather(idx_v, [chunk_rows])
            m_vec = plsc.load_gather(mask_v, [chunk_rows])
            picked = plsc.load_gather(
                logp_buf, [jnp.full((_L,), slot, jnp.int32), rowsel, t_vec])
            ml_acc = ml_acc + picked * m_vec * firstcopy
            for r in range(C):
                UNROLL = 8
                NACC = 4

                def body(j, carry):
                    zs = list(carry[:NACC])
                    ds_ = list(carry[NACC:])
                    base_off = j * (_L * UNROLL)
                    for u in range(UNROLL):
                        off = base_off + u * _L
                        s = sim_buf[slot, r, pl.ds(off, _L)]
                        lg = logp_buf[slot, r, pl.ds(off, _L)]
                        e = jnp.exp(s * inv_tau)
                        a = u % NACC
                        zs[a] = zs[a] + e
                        ds_[a] = ds_[a] + e * lg
                    return tuple(zs) + tuple(ds_)

                zero = jnp.zeros((_L,), jnp.float32)
                acc = lax.fori_loop(0, VCH // UNROLL, body, (zero,) * (2 * NACC))
                z = (acc[0] + acc[1]) + (acc[2] + acc[3])
                d = (acc[4] + acc[5]) + (acc[6] + acc[7])
                zt = _lane_total(z)
                dt = _lane_total(d)
                m = plsc.load_gather(
                    mask_v, [jnp.full((_L,), c * C + r, jnp.int32)])
                smooth_acc = smooth_acc + m * (dt / zt)

        msum_acc = jnp.zeros((_L,), jnp.float32)
        for g in range(RW // _L):
            msum_acc = msum_acc + mask_v[pl.ds(g * _L, _L)]

        out_stage[0, :] = smooth_acc
        out_stage[1, :] = ml_acc
        out_stage[2, :] = msum_acc
        pltpu.sync_copy(out_stage, out_hbm.at[wid])

    return sc_kernel


_N_SC = 1024


@jax.jit
def kernel(logp, sim_matrix, target, mask):
    B, T, V = logp.shape
    N = B * T
    flat_logp = logp.reshape(N, V)
    idx = target.reshape(N).astype(jnp.int32)
    flat_mask = mask.reshape(N)

    idx3 = idx[:_N_SC].reshape(_NW, -1, 4)
    tc = _make_tc_kernel(N, V, _N_SC)(idx, flat_mask,
                                      logp.reshape(N * V // 128, 128),
                                      sim_matrix.reshape(V * V // 128, 128))
    parts = _make_sc_kernel(_N_SC, V)(flat_logp, sim_matrix, idx, idx3,
                                      flat_mask)
    smooth_sum = jnp.sum(parts[:, 0, :]) / _L + tc[0, 0]
    ml_sum = jnp.sum(parts[:, 1, :]) + tc[0, 1]
    msum = jnp.sum(parts[:, 2, :]) + tc[0, 2]
    ml_output = -ml_sum / msum
    output = -smooth_sum / msum
    final = _ALPHA * output + (1.0 - _ALPHA) * ml_output
    return jnp.stack([final, ml_output])


# speedup vs baseline: 3.2171x; 3.2171x over previous
import functools

import jax
import jax.numpy as jnp
from jax import lax
from jax.experimental import pallas as pl
from jax.experimental.pallas import tpu as pltpu
from jax.experimental.pallas import tpu_sc as plsc

_TAU_WORD = 0.1
_ALPHA = 0.7

_NC = 2
_NS = 16
_L = 16
_NW = _NC * _NS

_GDN = lax.GatherDimensionNumbers(
    offset_dims=(), collapsed_slice_dims=(0,), start_index_map=(0,))


def _permute(x, idx):
    return lax.gather(x, idx[:, None], _GDN, (1,),
                      mode=lax.GatherScatterMode.PROMISE_IN_BOUNDS)


def _lane_total(x):
    iota = lax.iota(jnp.int32, _L)
    for shift in (8, 4, 2, 1):
        x = x + _permute(x, iota ^ shift)
    return x


def _make_tc_kernel(N, V, row0):
    R = 8
    NR = N - row0
    G = NR // R
    SL = V // 128

    def body(t_sref, m_sref, logp_hbm, sim_hbm, out_ref,
             sim_buf, logp_buf, acc_ref, sem_s, sem_l):
        i = pl.program_id(0)

        def issue(g, slot):
            row = row0 + g * R
            pltpu.make_async_copy(
                logp_hbm.at[pl.ds(row, R)], logp_buf.at[slot],
                sem_l.at[slot]).start()
            for k in range(R):
                t = t_sref[row + k]
                pltpu.make_async_copy(
                    sim_hbm.at[pl.ds(t, 1)], sim_buf.at[slot, pl.ds(k, 1)],
                    sem_s.at[slot]).start()

        @pl.when(i == 0)
        def _():
            acc_ref[...] = jnp.zeros_like(acc_ref)
            issue(0, 0)

        slot = i % 2

        @pl.when(i + 1 < G)
        def _():
            issue(i + 1, (i + 1) % 2)

        row = row0 + i * R
        pltpu.make_async_copy(
            logp_hbm.at[pl.ds(row, R)], logp_buf.at[slot],
            sem_l.at[slot]).wait()
        pltpu.make_async_copy(
            sim_hbm.at[pl.ds(0, R)], sim_buf.at[slot], sem_s.at[slot]).wait()

        lg = logp_buf[slot]
        e = jnp.exp(sim_buf[slot] * jnp.float32(1.0 / _TAU_WORD))
        z = jnp.sum(e, axis=1, keepdims=True)
        d = jnp.sum(e * lg, axis=1, keepdims=True)
        t_vec = jnp.stack(
            [t_sref[row + k] for k in range(R)])[:, None]
        m_vec = jnp.stack(
            [m_sref[row + k] for k in range(R)])[:, None]
        ci = lax.broadcasted_iota(jnp.int32, (R, V), 1)
        pick = jnp.sum(jnp.where(ci == t_vec, lg, 0.0), axis=1,
                       keepdims=True)
        li = lax.broadcasted_iota(jnp.int32, (R, 128), 1)
        contrib = jnp.where(
            li == 0, m_vec * d / z,
            jnp.where(li == 1, m_vec * pick,
                      jnp.where(li == 2, m_vec, 0.0)))
        acc_ref[...] += contrib

        @pl.when(i == G - 1)
        def _():
            out_ref[...] = jnp.sum(acc_ref[...], axis=0, keepdims=True)

    return pl.pallas_call(
        body,
        grid_spec=pltpu.PrefetchScalarGridSpec(
            num_scalar_prefetch=2,
            grid=(G,),
            in_specs=[
                pl.BlockSpec(memory_space=pltpu.HBM),
                pl.BlockSpec(memory_space=pltpu.HBM),
            ],
            out_specs=pl.BlockSpec((1, 128), lambda i, t, m: (0, 0)),
            scratch_shapes=[
                pltpu.VMEM((2, R, V), jnp.float32),
                pltpu.VMEM((2, R, V), jnp.float32),
                pltpu.VMEM((R, 128), jnp.float32),
                pltpu.SemaphoreType.DMA((2,)),
                pltpu.SemaphoreType.DMA((2,)),
            ],
        ),
        out_shape=jax.ShapeDtypeStruct((1, 128), jnp.float32),
    )


def _make_sc_kernel(N, V):
    RW = N // _NW
    C = 4
    NCHUNK = RW // C
    VCH = V // _L
    mesh = plsc.VectorSubcoreMesh(core_axis_name="c", subcore_axis_name="s",
                                  num_cores=_NC, num_subcores=_NS)

    @functools.partial(
        pl.kernel,
        mesh=mesh,
        compiler_params=pltpu.CompilerParams(needs_layout_passes=False),
        out_type=jax.ShapeDtypeStruct((_NW, 3, _L), jnp.float32),
        scratch_types=[
            pltpu.VMEM((RW,), jnp.int32),
            pltpu.VMEM((NCHUNK, C), jnp.int32),
            pltpu.VMEM((RW,), jnp.float32),
            pltpu.VMEM((2, C, V), jnp.float32),
            pltpu.VMEM((2, C, V), jnp.float32),
            pltpu.VMEM((3, _L), jnp.float32),
            pltpu.SemaphoreType.DMA,
            pltpu.SemaphoreType.DMA,
        ],
    )
    def sc_kernel(logp_hbm, sim_hbm, tgt_hbm, tgt3_hbm, mask_hbm,
                  out_hbm, idx_v, idx2_v, mask_v, sim_buf,
                  logp_buf, out_stage, sem_sim, sem_logp):
        wid = lax.axis_index("s") * _NC + lax.axis_index("c")
        base = wid * RW

        pltpu.sync_copy(tgt_hbm.at[pl.ds(base, RW)], idx_v)
        pltpu.sync_copy(tgt3_hbm.at[wid], idx2_v)
        pltpu.sync_copy(mask_hbm.at[pl.ds(base, RW)], mask_v)

        iota16 = lax.iota(jnp.int32, _L)
        rowsel = iota16 % C
        firstcopy = (iota16 < C).astype(jnp.float32)

        def start_chunk(c, slot):
            sim = pltpu.async_copy(
                sim_hbm.at[idx2_v.at[c]], sim_buf.at[slot], sem_sim)
            lp = pltpu.async_copy(
                logp_hbm.at[pl.ds(base + c * C, C)], logp_buf.at[slot],
                sem_logp)
            return sim, lp

        smooth_acc = jnp.zeros((_L,), jnp.float32)
        ml_acc = jnp.zeros((_L,), jnp.float32)
        inflight = start_chunk(0, 0)
        inv_tau = jnp.float32(1.0 / _TAU_WORD)

        for c in range(NCHUNK):
            slot = c % 2
            inflight[0].wait()
            inflight[1].wait()
            if c + 1 < NCHUNK:
                inflight = start_chunk(c + 1, (c + 1) % 2)
            chunk_rows = c * C + rowsel
            t_vec = plsc.load_gather(idx_v, [chunk_rows])
            m_vec = plsc.load_gather(mask_v, [chunk_rows])
            picked = plsc.load_gather(
                logp_buf, [jnp.full((_L,), slot, jnp.int32), rowsel, t_vec])
            ml_acc = ml_acc + picked * m_vec * firstcopy
            for r in range(C):
                UNROLL = 8
                NACC = 4

                def body(j, carry):
                    zs = list(carry[:NACC])
                    ds_ = list(carry[NACC:])
                    base_off = j * (_L * UNROLL)
                    for u in range(UNROLL):
                        off = base_off + u * _L
                        s = sim_buf[slot, r, pl.ds(off, _L)]
                        lg = logp_buf[slot, r, pl.ds(off, _L)]
                        e = jnp.exp(s * inv_tau)
                        a = u % NACC
                        zs[a] = zs[a] + e
                        ds_[a] = ds_[a] + e * lg
                    return tuple(zs) + tuple(ds_)

                zero = jnp.zeros((_L,), jnp.float32)
                acc = lax.fori_loop(0, VCH // UNROLL, body, (zero,) * (2 * NACC))
                z = (acc[0] + acc[1]) + (acc[2] + acc[3])
                d = (acc[4] + acc[5]) + (acc[6] + acc[7])
                zt = _lane_total(z)
                dt = _lane_total(d)
                m = plsc.load_gather(
                    mask_v, [jnp.full((_L,), c * C + r, jnp.int32)])
                smooth_acc = smooth_acc + m * (dt / zt)

        msum_acc = jnp.zeros((_L,), jnp.float32)
        for g in range(RW // _L):
            msum_acc = msum_acc + mask_v[pl.ds(g * _L, _L)]

        out_stage[0, :] = smooth_acc
        out_stage[1, :] = ml_acc
        out_stage[2, :] = msum_acc
        pltpu.sync_copy(out_stage, out_hbm.at[wid])

    return sc_kernel


_N_SC = 1536


@jax.jit
def kernel(logp, sim_matrix, target, mask):
    B, T, V = logp.shape
    N = B * T
    flat_logp = logp.reshape(N, V)
    idx = target.reshape(N).astype(jnp.int32)
    flat_mask = mask.reshape(N)

    idx3 = idx[:_N_SC].reshape(_NW, -1, 4)
    tc = _make_tc_kernel(N, V, _N_SC)(idx, flat_mask, flat_logp, sim_matrix)
    parts = _make_sc_kernel(_N_SC, V)(flat_logp, sim_matrix, idx, idx3,
                                      flat_mask)
    smooth_sum = jnp.sum(parts[:, 0, :]) / _L + tc[0, 0]
    ml_sum = jnp.sum(parts[:, 1, :]) + tc[0, 1]
    msum = jnp.sum(parts[:, 2, :]) + tc[0, 2]
    ml_output = -ml_sum / msum
    output = -smooth_sum / msum
    final = _ALPHA * output + (1.0 - _ALPHA) * ml_output
    return jnp.stack([final, ml_output])


# speedup vs baseline: 3.3666x; 1.0465x over previous
import functools

import jax
import jax.numpy as jnp
from jax import lax
from jax.experimental import pallas as pl
from jax.experimental.pallas import tpu as pltpu
from jax.experimental.pallas import tpu_sc as plsc

_TAU_WORD = 0.1
_ALPHA = 0.7

_NC = 2
_NS = 16
_L = 16
_NW = _NC * _NS

_GDN = lax.GatherDimensionNumbers(
    offset_dims=(), collapsed_slice_dims=(0,), start_index_map=(0,))


def _permute(x, idx):
    return lax.gather(x, idx[:, None], _GDN, (1,),
                      mode=lax.GatherScatterMode.PROMISE_IN_BOUNDS)


def _lane_total(x):
    iota = lax.iota(jnp.int32, _L)
    for shift in (8, 4, 2, 1):
        x = x + _permute(x, iota ^ shift)
    return x


def _make_tc_kernel(N, V, row0):
    R = 8
    NR = N - row0
    G = NR // R
    SL = V // 128

    def body(t_sref, m_sref, logp_hbm, sim_hbm, out_ref,
             sim_buf, logp_buf, acc_ref, sem_s, sem_l):
        i = pl.program_id(0)

        def issue(g, slot):
            row = row0 + g * R
            pltpu.make_async_copy(
                logp_hbm.at[pl.ds(row, R)], logp_buf.at[slot],
                sem_l.at[slot]).start()
            for k in range(R):
                t = t_sref[row + k]
                pltpu.make_async_copy(
                    sim_hbm.at[pl.ds(t, 1)], sim_buf.at[slot, pl.ds(k, 1)],
                    sem_s.at[slot]).start()

        @pl.when(i == 0)
        def _():
            acc_ref[...] = jnp.zeros_like(acc_ref)
            issue(0, 0)

        slot = i % 2

        @pl.when(i + 1 < G)
        def _():
            issue(i + 1, (i + 1) % 2)

        row = row0 + i * R
        pltpu.make_async_copy(
            logp_hbm.at[pl.ds(row, R)], logp_buf.at[slot],
            sem_l.at[slot]).wait()
        pltpu.make_async_copy(
            sim_hbm.at[pl.ds(0, R)], sim_buf.at[slot], sem_s.at[slot]).wait()

        lg = logp_buf[slot]
        e = jnp.exp(sim_buf[slot] * jnp.float32(1.0 / _TAU_WORD))
        z = jnp.sum(e, axis=1, keepdims=True)
        d = jnp.sum(e * lg, axis=1, keepdims=True)
        t_vec = jnp.stack(
            [t_sref[row + k] for k in range(R)])[:, None]
        m_vec = jnp.stack(
            [m_sref[row + k] for k in range(R)])[:, None]
        ci = lax.broadcasted_iota(jnp.int32, (R, V), 1)
        pick = jnp.sum(jnp.where(ci == t_vec, lg, 0.0), axis=1,
                       keepdims=True)
        li = lax.broadcasted_iota(jnp.int32, (R, 128), 1)
        contrib = jnp.where(
            li == 0, m_vec * d / z,
            jnp.where(li == 1, m_vec * pick,
                      jnp.where(li == 2, m_vec, 0.0)))
        acc_ref[...] += contrib

        @pl.when(i == G - 1)
        def _():
            out_ref[...] = jnp.sum(acc_ref[...], axis=0, keepdims=True)

    return pl.pallas_call(
        body,
        grid_spec=pltpu.PrefetchScalarGridSpec(
            num_scalar_prefetch=2,
            grid=(G,),
            in_specs=[
                pl.BlockSpec(memory_space=pltpu.HBM),
                pl.BlockSpec(memory_space=pltpu.HBM),
            ],
            out_specs=pl.BlockSpec((1, 128), lambda i, t, m: (0, 0)),
            scratch_shapes=[
                pltpu.VMEM((2, R, V), jnp.float32),
                pltpu.VMEM((2, R, V), jnp.float32),
                pltpu.VMEM((R, 128), jnp.float32),
                pltpu.SemaphoreType.DMA((2,)),
                pltpu.SemaphoreType.DMA((2,)),
            ],
        ),
        out_shape=jax.ShapeDtypeStruct((1, 128), jnp.float32),
    )


def _make_sc_kernel(N, V):
    RW = N // _NW
    C = 4
    NCHUNK = RW // C
    VCH = V // _L
    mesh = plsc.VectorSubcoreMesh(core_axis_name="c", subcore_axis_name="s",
                                  num_cores=_NC, num_subcores=_NS)

    @functools.partial(
        pl.kernel,
        mesh=mesh,
        compiler_params=pltpu.CompilerParams(needs_layout_passes=False),
        out_type=jax.ShapeDtypeStruct((_NW, 3, _L), jnp.float32),
        scratch_types=[
            pltpu.VMEM((RW,), jnp.int32),
            pltpu.VMEM((NCHUNK, C), jnp.int32),
            pltpu.VMEM((RW,), jnp.float32),
            pltpu.VMEM((2, C, V), jnp.float32),
            pltpu.VMEM((2, C, V), jnp.float32),
            pltpu.VMEM((3, _L), jnp.float32),
            pltpu.SemaphoreType.DMA,
            pltpu.SemaphoreType.DMA,
        ],
    )
    def sc_kernel(logp_hbm, sim_hbm, tgt_hbm, tgt3_hbm, mask_hbm,
                  out_hbm, idx_v, idx2_v, mask_v, sim_buf,
                  logp_buf, out_stage, sem_sim, sem_logp):
        wid = lax.axis_index("s") * _NC + lax.axis_index("c")
        base = wid * RW

        pltpu.sync_copy(tgt_hbm.at[pl.ds(base, RW)], idx_v)
        pltpu.sync_copy(tgt3_hbm.at[wid], idx2_v)
        pltpu.sync_copy(mask_hbm.at[pl.ds(base, RW)], mask_v)

        iota16 = lax.iota(jnp.int32, _L)
        rowsel = iota16 % C
        firstcopy = (iota16 < C).astype(jnp.float32)

        def start_chunk(c, slot):
            sim = pltpu.async_copy(
                sim_hbm.at[idx2_v.at[c]], sim_buf.at[slot], sem_sim)
            lp = pltpu.async_copy(
                logp_hbm.at[pl.ds(base + c * C, C)], logp_buf.at[slot],
                sem_logp)
            return sim, lp

        smooth_acc = jnp.zeros((_L,), jnp.float32)
        ml_acc = jnp.zeros((_L,), jnp.float32)
        msum_acc = jnp.zeros((_L,), jnp.float32)
        inflight = start_chunk(0, 0)
        inv_tau = jnp.float32(1.0 / _TAU_WORD)

        for c in range(NCHUNK):
            slot = c % 2
            inflight[0].wait()
            inflight[1].wait()
            if c + 1 < NCHUNK:
                inflight = start_chunk(c + 1, (c + 1) % 2)
            chunk_rows = c * C + rowsel
            t_vec = plsc.load_gather(idx_v, [chunk_rows])
            m_vec = plsc.load_gather(mask_v, [chunk_rows])
            picked = plsc.load_gather(
                logp_buf, [jnp.full((_L,), slot, jnp.int32), rowsel, t_vec])
            ml_acc = ml_acc + picked * m_vec * firstcopy
            msum_acc = msum_acc + m_vec * firstcopy
            for r in range(C):
                UNROLL = 8
                NACC = 4

                def body(j, carry):
                    zs = list(carry[:NACC])
                    ds_ = list(carry[NACC:])
                    base_off = j * (_L * UNROLL)
                    for u in range(UNROLL):
                        off = base_off + u * _L
                        s = sim_buf[slot, r, pl.ds(off, _L)]
                        lg = logp_buf[slot, r, pl.ds(off, _L)]
                        e = jnp.exp(s * inv_tau)
                        a = u % NACC
                        zs[a] = zs[a] + e
                        ds_[a] = ds_[a] + e * lg
                    return tuple(zs) + tuple(ds_)

                zero = jnp.zeros((_L,), jnp.float32)
                acc = lax.fori_loop(0, VCH // UNROLL, body, (zero,) * (2 * NACC))
                z = (acc[0] + acc[1]) + (acc[2] + acc[3])
                d = (acc[4] + acc[5]) + (acc[6] + acc[7])
                zt = _lane_total(z)
                dt = _lane_total(d)
                m = plsc.load_gather(
                    mask_v, [jnp.full((_L,), c * C + r, jnp.int32)])
                smooth_acc = smooth_acc + m * (dt / zt)

        out_stage[0, :] = smooth_acc
        out_stage[1, :] = ml_acc
        out_stage[2, :] = msum_acc
        pltpu.sync_copy(out_stage, out_hbm.at[wid])

    return sc_kernel


_N_SC = 1792


@jax.jit
def kernel(logp, sim_matrix, target, mask):
    B, T, V = logp.shape
    N = B * T
    flat_logp = logp.reshape(N, V)
    idx = target.reshape(N).astype(jnp.int32)
    flat_mask = mask.reshape(N)

    idx3 = idx[:_N_SC].reshape(_NW, -1, 4)
    tc = _make_tc_kernel(N, V, _N_SC)(idx, flat_mask, flat_logp, sim_matrix)
    parts = _make_sc_kernel(_N_SC, V)(flat_logp, sim_matrix, idx, idx3,
                                      flat_mask)
    smooth_sum = jnp.sum(parts[:, 0, :]) / _L + tc[0, 0]
    ml_sum = jnp.sum(parts[:, 1, :]) + tc[0, 1]
    msum = jnp.sum(parts[:, 2, :]) + tc[0, 2]
    ml_output = -ml_sum / msum
    output = -smooth_sum / msum
    final = _ALPHA * output + (1.0 - _ALPHA) * ml_output
    return jnp.stack([final, ml_output])


# speedup vs baseline: 3.3814x; 1.0044x over previous
import functools

import jax
import jax.numpy as jnp
from jax import lax
from jax.experimental import pallas as pl
from jax.experimental.pallas import tpu as pltpu
from jax.experimental.pallas import tpu_sc as plsc

_TAU_WORD = 0.1
_ALPHA = 0.7

_NC = 2
_NS = 16
_L = 16
_NW = _NC * _NS

_GDN = lax.GatherDimensionNumbers(
    offset_dims=(), collapsed_slice_dims=(0,), start_index_map=(0,))


def _permute(x, idx):
    return lax.gather(x, idx[:, None], _GDN, (1,),
                      mode=lax.GatherScatterMode.PROMISE_IN_BOUNDS)


def _lane_total(x):
    iota = lax.iota(jnp.int32, _L)
    for shift in (8, 4, 2, 1):
        x = x + _permute(x, iota ^ shift)
    return x


def _make_tc_kernel(N, V, row0):
    R = 8
    NR = N - row0
    G = NR // R
    SL = V // 128

    def body(t_sref, m_sref, logp_hbm, sim_hbm, out_ref,
             sim_buf, logp_buf, acc_ref, sem_s, sem_l):
        i = pl.program_id(0)

        def issue(g, slot):
            row = row0 + g * R
            pltpu.make_async_copy(
                logp_hbm.at[pl.ds(row, R)], logp_buf.at[slot],
                sem_l.at[slot]).start()
            for k in range(R):
                t = t_sref[row + k]
                pltpu.make_async_copy(
                    sim_hbm.at[pl.ds(t, 1)], sim_buf.at[slot, pl.ds(k, 1)],
                    sem_s.at[slot]).start()

        @pl.when(i == 0)
        def _():
            acc_ref[...] = jnp.zeros_like(acc_ref)
            issue(0, 0)
            issue(1, 1)

        slot = i % 3

        @pl.when(i + 2 < G)
        def _():
            issue(i + 2, (i + 2) % 3)

        row = row0 + i * R
        pltpu.make_async_copy(
            logp_hbm.at[pl.ds(row, R)], logp_buf.at[slot],
            sem_l.at[slot]).wait()
        pltpu.make_async_copy(
            sim_hbm.at[pl.ds(0, R)], sim_buf.at[slot], sem_s.at[slot]).wait()

        lg = logp_buf[slot]
        e = jnp.exp(sim_buf[slot] * jnp.float32(1.0 / _TAU_WORD))
        z = jnp.sum(e, axis=1, keepdims=True)
        d = jnp.sum(e * lg, axis=1, keepdims=True)
        t_vec = jnp.stack(
            [t_sref[row + k] for k in range(R)])[:, None]
        m_vec = jnp.stack(
            [m_sref[row + k] for k in range(R)])[:, None]
        ci = lax.broadcasted_iota(jnp.int32, (R, V), 1)
        pick = jnp.sum(jnp.where(ci == t_vec, lg, 0.0), axis=1,
                       keepdims=True)
        li = lax.broadcasted_iota(jnp.int32, (R, 128), 1)
        contrib = jnp.where(
            li == 0, m_vec * d / z,
            jnp.where(li == 1, m_vec * pick,
                      jnp.where(li == 2, m_vec, 0.0)))
        acc_ref[...] += contrib

        @pl.when(i == G - 1)
        def _():
            out_ref[...] = jnp.sum(acc_ref[...], axis=0, keepdims=True)

    return pl.pallas_call(
        body,
        grid_spec=pltpu.PrefetchScalarGridSpec(
            num_scalar_prefetch=2,
            grid=(G,),
            in_specs=[
                pl.BlockSpec(memory_space=pltpu.HBM),
                pl.BlockSpec(memory_space=pltpu.HBM),
            ],
            out_specs=pl.BlockSpec((1, 128), lambda i, t, m: (0, 0)),
            scratch_shapes=[
                pltpu.VMEM((3, R, V), jnp.float32),
                pltpu.VMEM((3, R, V), jnp.float32),
                pltpu.VMEM((R, 128), jnp.float32),
                pltpu.SemaphoreType.DMA((3,)),
                pltpu.SemaphoreType.DMA((3,)),
            ],
        ),
        out_shape=jax.ShapeDtypeStruct((1, 128), jnp.float32),
    )


def _make_sc_kernel(N, V):
    RW = N // _NW
    C = 4
    NCHUNK = RW // C
    VCH = V // _L
    mesh = plsc.VectorSubcoreMesh(core_axis_name="c", subcore_axis_name="s",
                                  num_cores=_NC, num_subcores=_NS)

    @functools.partial(
        pl.kernel,
        mesh=mesh,
        compiler_params=pltpu.CompilerParams(needs_layout_passes=False),
        out_type=jax.ShapeDtypeStruct((_NW, 3, _L), jnp.float32),
        scratch_types=[
            pltpu.VMEM((RW,), jnp.int32),
            pltpu.VMEM((NCHUNK, C), jnp.int32),
            pltpu.VMEM((RW,), jnp.float32),
            pltpu.VMEM((2, C, V), jnp.float32),
            pltpu.VMEM((2, C, V), jnp.float32),
            pltpu.VMEM((3, _L), jnp.float32),
            pltpu.SemaphoreType.DMA,
            pltpu.SemaphoreType.DMA,
        ],
    )
    def sc_kernel(logp_hbm, sim_hbm, tgt_hbm, tgt3_hbm, mask_hbm,
                  out_hbm, idx_v, idx2_v, mask_v, sim_buf,
                  logp_buf, out_stage, sem_sim, sem_logp):
        wid = lax.axis_index("s") * _NC + lax.axis_index("c")
        base = wid * RW

        pltpu.sync_copy(tgt_hbm.at[pl.ds(base, RW)], idx_v)
        pltpu.sync_copy(tgt3_hbm.at[wid], idx2_v)
        pltpu.sync_copy(mask_hbm.at[pl.ds(base, RW)], mask_v)

        iota16 = lax.iota(jnp.int32, _L)
        rowsel = iota16 % C
        firstcopy = (iota16 < C).astype(jnp.float32)

        def start_chunk(c, slot):
            sim = pltpu.async_copy(
                sim_hbm.at[idx2_v.at[c]], sim_buf.at[slot], sem_sim)
            lp = pltpu.async_copy(
                logp_hbm.at[pl.ds(base + c * C, C)], logp_buf.at[slot],
                sem_logp)
            return sim, lp

        smooth_acc = jnp.zeros((_L,), jnp.float32)
        ml_acc = jnp.zeros((_L,), jnp.float32)
        msum_acc = jnp.zeros((_L,), jnp.float32)
        inflight = start_chunk(0, 0)
        inv_tau = jnp.float32(1.0 / _TAU_WORD)

        for c in range(NCHUNK):
            slot = c % 2
            inflight[0].wait()
            inflight[1].wait()
            if c + 1 < NCHUNK:
                inflight = start_chunk(c + 1, (c + 1) % 2)
            chunk_rows = c * C + rowsel
            t_vec = plsc.load_gather(idx_v, [chunk_rows])
            m_vec = plsc.load_gather(mask_v, [chunk_rows])
            picked = plsc.load_gather(
                logp_buf, [jnp.full((_L,), slot, jnp.int32), rowsel, t_vec])
            ml_acc = ml_acc + picked * m_vec * firstcopy
            msum_acc = msum_acc + m_vec * firstcopy
            for r in range(C):
                UNROLL = 8
                NACC = 4

                def body(j, carry):
                    zs = list(carry[:NACC])
                    ds_ = list(carry[NACC:])
                    base_off = j * (_L * UNROLL)
                    for u in range(UNROLL):
                        off = base_off + u * _L
                        s = sim_buf[slot, r, pl.ds(off, _L)]
                        lg = logp_buf[slot, r, pl.ds(off, _L)]
                        e = jnp.exp(s * inv_tau)
                        a = u % NACC
                        zs[a] = zs[a] + e
                        ds_[a] = ds_[a] + e * lg
                    return tuple(zs) + tuple(ds_)

                zero = jnp.zeros((_L,), jnp.float32)
                acc = lax.fori_loop(0, VCH // UNROLL, body, (zero,) * (2 * NACC))
                z = (acc[0] + acc[1]) + (acc[2] + acc[3])
                d = (acc[4] + acc[5]) + (acc[6] + acc[7])
                zt = _lane_total(z)
                dt = _lane_total(d)
                m = plsc.load_gather(
                    mask_v, [jnp.full((_L,), c * C + r, jnp.int32)])
                smooth_acc = smooth_acc + m * (dt / zt)

        out_stage[0, :] = smooth_acc
        out_stage[1, :] = ml_acc
        out_stage[2, :] = msum_acc
        pltpu.sync_copy(out_stage, out_hbm.at[wid])

    return sc_kernel


_N_SC = 1792


@jax.jit
def kernel(logp, sim_matrix, target, mask):
    B, T, V = logp.shape
    N = B * T
    flat_logp = logp.reshape(N, V)
    idx = target.reshape(N).astype(jnp.int32)
    flat_mask = mask.reshape(N)

    idx3 = idx[:_N_SC].reshape(_NW, -1, 4)
    tc = _make_tc_kernel(N, V, _N_SC)(idx, flat_mask, flat_logp, sim_matrix)
    parts = _make_sc_kernel(_N_SC, V)(flat_logp, sim_matrix, idx, idx3,
                                      flat_mask)
    smooth_sum = jnp.sum(parts[:, 0, :]) / _L + tc[0, 0]
    ml_sum = jnp.sum(parts[:, 1, :]) + tc[0, 1]
    msum = jnp.sum(parts[:, 2, :]) + tc[0, 2]
    ml_output = -ml_sum / msum
    output = -smooth_sum / msum
    final = _ALPHA * output + (1.0 - _ALPHA) * ml_output
    return jnp.stack([final, ml_output])


# speedup vs baseline: 3.6332x; 1.0745x over previous
import functools

import jax
import jax.numpy as jnp
from jax import lax
from jax.experimental import pallas as pl
from jax.experimental.pallas import tpu as pltpu
from jax.experimental.pallas import tpu_sc as plsc

_TAU_WORD = 0.1
_ALPHA = 0.7

_NC = 2
_NS = 16
_L = 16
_NW = _NC * _NS

_GDN = lax.GatherDimensionNumbers(
    offset_dims=(), collapsed_slice_dims=(0,), start_index_map=(0,))


def _permute(x, idx):
    return lax.gather(x, idx[:, None], _GDN, (1,),
                      mode=lax.GatherScatterMode.PROMISE_IN_BOUNDS)


def _lane_total(x):
    iota = lax.iota(jnp.int32, _L)
    for shift in (8, 4, 2, 1):
        x = x + _permute(x, iota ^ shift)
    return x


def _make_tc_kernel(N, V, row0):
    R = 8
    NR = N - row0
    G = NR // R
    SL = V // 128

    def body(t_sref, m_sref, logp_hbm, sim_hbm, out_ref,
             sim_buf, logp_buf, acc_ref, sem_s, sem_l):
        i = pl.program_id(0)

        def issue(g, slot):
            row = row0 + g * R
            pltpu.make_async_copy(
                logp_hbm.at[pl.ds(row, R)], logp_buf.at[slot],
                sem_l.at[slot]).start()
            for k in range(R):
                t = t_sref[row + k]
                pltpu.make_async_copy(
                    sim_hbm.at[pl.ds(t, 1)], sim_buf.at[slot, pl.ds(k, 1)],
                    sem_s.at[slot]).start()

        @pl.when(i == 0)
        def _():
            acc_ref[...] = jnp.zeros_like(acc_ref)
            issue(0, 0)
            issue(1, 1)

        slot = i % 3

        @pl.when(i + 2 < G)
        def _():
            issue(i + 2, (i + 2) % 3)

        row = row0 + i * R
        pltpu.make_async_copy(
            logp_hbm.at[pl.ds(row, R)], logp_buf.at[slot],
            sem_l.at[slot]).wait()
        pltpu.make_async_copy(
            sim_hbm.at[pl.ds(0, R)], sim_buf.at[slot], sem_s.at[slot]).wait()

        lg = logp_buf[slot]
        e = jnp.exp(sim_buf[slot] * jnp.float32(1.0 / _TAU_WORD))
        z = jnp.sum(e, axis=1, keepdims=True)
        d = jnp.sum(e * lg, axis=1, keepdims=True)
        t_vec = jnp.stack(
            [t_sref[row + k] for k in range(R)])[:, None]
        m_vec = jnp.stack(
            [m_sref[row + k] for k in range(R)])[:, None]
        ci = lax.broadcasted_iota(jnp.int32, (R, V), 1)
        pick = jnp.sum(jnp.where(ci == t_vec, lg, 0.0), axis=1,
                       keepdims=True)
        li = lax.broadcasted_iota(jnp.int32, (R, 128), 1)
        contrib = jnp.where(
            li == 0, m_vec * d / z,
            jnp.where(li == 1, m_vec * pick,
                      jnp.where(li == 2, m_vec, 0.0)))
        acc_ref[...] += contrib

        @pl.when(i == G - 1)
        def _():
            out_ref[...] = jnp.sum(acc_ref[...], axis=0, keepdims=True)

    return pl.pallas_call(
        body,
        grid_spec=pltpu.PrefetchScalarGridSpec(
            num_scalar_prefetch=2,
            grid=(G,),
            in_specs=[
                pl.BlockSpec(memory_space=pltpu.HBM),
                pl.BlockSpec(memory_space=pltpu.HBM),
            ],
            out_specs=pl.BlockSpec((1, 128), lambda i, t, m: (0, 0)),
            scratch_shapes=[
                pltpu.VMEM((3, R, V), jnp.float32),
                pltpu.VMEM((3, R, V), jnp.float32),
                pltpu.VMEM((R, 128), jnp.float32),
                pltpu.SemaphoreType.DMA((3,)),
                pltpu.SemaphoreType.DMA((3,)),
            ],
        ),
        out_shape=jax.ShapeDtypeStruct((1, 128), jnp.float32),
    )


def _make_sc_kernel(N, V):
    RW = N // _NW
    C = 4
    NCHUNK = RW // C
    VCH = V // _L
    mesh = plsc.VectorSubcoreMesh(core_axis_name="c", subcore_axis_name="s",
                                  num_cores=_NC, num_subcores=_NS)

    @functools.partial(
        pl.kernel,
        mesh=mesh,
        compiler_params=pltpu.CompilerParams(needs_layout_passes=False),
        out_type=jax.ShapeDtypeStruct((_NW, 3, _L), jnp.float32),
        scratch_types=[
            pltpu.VMEM((RW,), jnp.int32),
            pltpu.VMEM((NCHUNK, C), jnp.int32),
            pltpu.VMEM((RW,), jnp.float32),
            pltpu.VMEM((2, C, V), jnp.float32),
            pltpu.VMEM((2, C, V), jnp.float32),
            pltpu.VMEM((3, _L), jnp.float32),
            pltpu.SemaphoreType.DMA,
            pltpu.SemaphoreType.DMA,
        ],
    )
    def sc_kernel(logp_hbm, sim_hbm, tgt_hbm, tgt3_hbm, mask_hbm,
                  out_hbm, idx_v, idx2_v, mask_v, sim_buf,
                  logp_buf, out_stage, sem_sim, sem_logp):
        wid = lax.axis_index("s") * _NC + lax.axis_index("c")
        base = wid * RW

        pltpu.sync_copy(tgt_hbm.at[pl.ds(base, RW)], idx_v)
        pltpu.sync_copy(tgt3_hbm.at[wid], idx2_v)
        pltpu.sync_copy(mask_hbm.at[pl.ds(base, RW)], mask_v)

        iota16 = lax.iota(jnp.int32, _L)
        rowsel = iota16 % C
        firstcopy = (iota16 < C).astype(jnp.float32)

        def start_chunk(c, slot):
            sim = pltpu.async_copy(
                sim_hbm.at[idx2_v.at[c]], sim_buf.at[slot], sem_sim)
            lp = pltpu.async_copy(
                logp_hbm.at[pl.ds(base + c * C, C)], logp_buf.at[slot],
                sem_logp)
            return sim, lp

        smooth_acc = jnp.zeros((_L,), jnp.float32)
        ml_acc = jnp.zeros((_L,), jnp.float32)
        msum_acc = jnp.zeros((_L,), jnp.float32)
        inflight = start_chunk(0, 0)
        inv_tau = jnp.float32(1.0 / _TAU_WORD)

        for c in range(NCHUNK):
            slot = c % 2
            inflight[0].wait()
            inflight[1].wait()
            if c + 1 < NCHUNK:
                inflight = start_chunk(c + 1, (c + 1) % 2)
            chunk_rows = c * C + rowsel
            t_vec = plsc.load_gather(idx_v, [chunk_rows])
            m_vec = plsc.load_gather(mask_v, [chunk_rows])
            picked = plsc.load_gather(
                logp_buf, [jnp.full((_L,), slot, jnp.int32), rowsel, t_vec])
            ml_acc = ml_acc + picked * m_vec * firstcopy
            msum_acc = msum_acc + m_vec * firstcopy
            for r in range(C):
                UNROLL = 8
                NACC = 4

                def body(j, carry):
                    zs = list(carry[:NACC])
                    ds_ = list(carry[NACC:])
                    base_off = j * (_L * UNROLL)
                    for u in range(UNROLL):
                        off = base_off + u * _L
                        s = sim_buf[slot, r, pl.ds(off, _L)]
                        lg = logp_buf[slot, r, pl.ds(off, _L)]
                        e = jnp.exp(s * inv_tau)
                        a = u % NACC
                        zs[a] = zs[a] + e
                        ds_[a] = ds_[a] + e * lg
                    return tuple(zs) + tuple(ds_)

                zero = jnp.zeros((_L,), jnp.float32)
                acc = lax.fori_loop(0, VCH // UNROLL, body, (zero,) * (2 * NACC))
                z = (acc[0] + acc[1]) + (acc[2] + acc[3])
                d = (acc[4] + acc[5]) + (acc[6] + acc[7])
                zt = _lane_total(z)
                dt = _lane_total(d)
                m = plsc.load_gather(
                    mask_v, [jnp.full((_L,), c * C + r, jnp.int32)])
                smooth_acc = smooth_acc + m * (dt / zt)

        out_stage[0, :] = smooth_acc
        out_stage[1, :] = ml_acc
        out_stage[2, :] = msum_acc
        pltpu.sync_copy(out_stage, out_hbm.at[wid])

    return sc_kernel


_N_SC = 1536


@jax.jit
def kernel(logp, sim_matrix, target, mask):
    B, T, V = logp.shape
    N = B * T
    flat_logp = logp.reshape(N, V)
    idx = target.reshape(N).astype(jnp.int32)
    flat_mask = mask.reshape(N)

    idx3 = idx[:_N_SC].reshape(_NW, -1, 4)
    tc = _make_tc_kernel(N, V, _N_SC)(idx, flat_mask, flat_logp, sim_matrix)
    parts = _make_sc_kernel(_N_SC, V)(flat_logp, sim_matrix, idx, idx3,
                                      flat_mask)
    smooth_sum = jnp.sum(parts[:, 0, :]) / _L + tc[0, 0]
    ml_sum = jnp.sum(parts[:, 1, :]) + tc[0, 1]
    msum = jnp.sum(parts[:, 2, :]) + tc[0, 2]
    ml_output = -ml_sum / msum
    output = -smooth_sum / msum
    final = _ALPHA * output + (1.0 - _ALPHA) * ml_output
    return jnp.stack([final, ml_output])
